# baseline (speedup 1.0000x reference)
import functools

import jax
import jax.numpy as jnp
from jax import lax
from jax.experimental import pallas as pl
from jax.experimental.pallas import tpu as pltpu
from jax.experimental.pallas import tpu_sc as plsc

HIDDEN = 4096
NUM_EXPERTS = 64
TOP_K = 8
LOAD_BALANCE_COEF = 0.001
BLOCK_T = 1024
N_WORKERS = 32
GROUPS = 64


def _matmul_block(x_ref, wt_ref, b_ref, lt_ref):
    xb = x_ref[...].astype(jnp.bfloat16)
    logits = jnp.dot(xb, wt_ref[...], preferred_element_type=jnp.float32)
    logits = logits + b_ref[...]
    lt_ref[...] = logits.T


def _tc_logits_t(x2, wt, b2, n_tokens):
    n_blocks = n_tokens // BLOCK_T
    return pl.pallas_call(
        _matmul_block,
        grid=(n_blocks,),
        in_specs=[
            pl.BlockSpec((BLOCK_T, HIDDEN), lambda i: (i, 0)),
            pl.BlockSpec((HIDDEN, NUM_EXPERTS), lambda i: (0, 0)),
            pl.BlockSpec((1, NUM_EXPERTS), lambda i: (0, 0)),
        ],
        out_specs=pl.BlockSpec((NUM_EXPERTS, BLOCK_T), lambda i: (0, i)),
        out_shape=jax.ShapeDtypeStruct((NUM_EXPERTS, n_tokens), jnp.float32),
    )(x2, wt, b2)


def _sc_route(lt_hbm, rw_hbm, se_hbm, f_hbm, p_hbm,
              buf, etab, rwb, seb, accf, accp):
    wid = lax.axis_index("s") * 2 + lax.axis_index("c")
    base = wid * BLOCK_T
    pltpu.sync_copy(lt_hbm.at[:, pl.ds(base, BLOCK_T)], buf)

    iota16 = lax.iota(jnp.int32, 16)
    zeros16 = jnp.zeros((16,), jnp.float32)
    ones16 = jnp.ones((16,), jnp.float32)

    def zero_body(e, _):
        accf[e, :] = zeros16
        accp[e, :] = zeros16
        return 0
    lax.fori_loop(0, NUM_EXPERTS, zero_body, 0)

    def group_body(g, _):
        col = g * 16
        m0 = buf[0, pl.ds(col, 16)]
        for e in range(1, NUM_EXPERTS):
            m0 = jnp.maximum(m0, buf[e, pl.ds(col, 16)])
        s = zeros16
        for e in range(NUM_EXPERTS):
            t = jnp.exp(buf[e, pl.ds(col, 16)] - m0)
            etab[e, :] = t
            s = s + t
        rcp = ones16 / s
        for e in range(NUM_EXPERTS):
            accp[e, :] = accp[e, :] + etab[e, :] * rcp
        top_e = []
        top_i = []
        for _ in range(TOP_K):
            m = etab[0, :]
            mi = jnp.zeros((16,), jnp.int32)
            for e in range(1, NUM_EXPERTS):
                v = etab[e, :]
                c = v > m
                m = jnp.where(c, v, m)
                mi = jnp.where(c, e, mi)
            top_e.append(m)
            top_i.append(mi)
            plsc.store_scatter(etab, [mi, iota16], zeros16)
        plsc.addupdate_scatter(accf, [top_i[0], iota16], ones16)
        s8 = top_e[0]
        for j in range(1, TOP_K):
            s8 = s8 + top_e[j]
        rcp8 = ones16 / s8
        for j in range(TOP_K):
            rwb[j, pl.ds(col, 16)] = top_e[j] * rcp8
            seb[j, pl.ds(col, 16)] = top_i[j]
        return 0

    lax.fori_loop(0, GROUPS, group_body, 0)

    pltpu.sync_copy(rwb, rw_hbm.at[:, pl.ds(base, BLOCK_T)])
    pltpu.sync_copy(seb, se_hbm.at[:, pl.ds(base, BLOCK_T)])
    pltpu.sync_copy(accf, f_hbm.at[wid])
    pltpu.sync_copy(accp, p_hbm.at[wid])


def _sc_route_call(lt, n_tokens):
    mesh = plsc.VectorSubcoreMesh(core_axis_name="c", subcore_axis_name="s")
    fn = pl.kernel(
        _sc_route,
        mesh=mesh,
        out_type=[
            jax.ShapeDtypeStruct((TOP_K, n_tokens), jnp.float32),
            jax.ShapeDtypeStruct((TOP_K, n_tokens), jnp.int32),
            jax.ShapeDtypeStruct((N_WORKERS, NUM_EXPERTS, 16), jnp.float32),
            jax.ShapeDtypeStruct((N_WORKERS, NUM_EXPERTS, 16), jnp.float32),
        ],
        scratch_types=[
            pltpu.VMEM((NUM_EXPERTS, BLOCK_T), jnp.float32),
            pltpu.VMEM((NUM_EXPERTS, 16), jnp.float32),
            pltpu.VMEM((TOP_K, BLOCK_T), jnp.float32),
            pltpu.VMEM((TOP_K, BLOCK_T), jnp.int32),
            pltpu.VMEM((NUM_EXPERTS, 16), jnp.float32),
            pltpu.VMEM((NUM_EXPERTS, 16), jnp.float32),
        ],
        compiler_params=pltpu.CompilerParams(needs_layout_passes=False,
                                             use_tc_tiling_on_sc=False),
    )
    return fn(lt)


def _aux_kernel(f_ref, p_ref, aux_ref, *, n_tokens):
    f_tot = jnp.sum(f_ref[...], axis=(0, 2))
    p_tot = jnp.sum(p_ref[...], axis=(0, 2))
    scale = NUM_EXPERTS * LOAD_BALANCE_COEF / (n_tokens * n_tokens)
    aux_ref[...] = (scale * jnp.sum(f_tot * p_tot)).reshape(1, 1)


def kernel(x, W, b):
    bsz, seq, hidden = x.shape
    n_tokens = bsz * seq
    x2 = x.reshape(n_tokens, hidden)
    wt = W.T.astype(jnp.bfloat16)
    b2 = b.reshape(1, NUM_EXPERTS)

    lt = _tc_logits_t(x2, wt, b2, n_tokens)
    rw, se, f_parts, p_parts = _sc_route_call(lt, n_tokens)

    aux = pl.pallas_call(
        functools.partial(_aux_kernel, n_tokens=n_tokens),
        out_shape=jax.ShapeDtypeStruct((1, 1), jnp.float32),
    )(f_parts, p_parts)

    return (rw.T.reshape(bsz, seq, TOP_K),
            se.T.reshape(bsz, seq, TOP_K),
            aux.reshape(()))


# speedup vs baseline: 2.0554x; 2.0554x over previous
import functools

import jax
import jax.numpy as jnp
from jax.experimental import pallas as pl
from jax.experimental.pallas import tpu as pltpu

HIDDEN = 4096
NUM_EXPERTS = 64
TOP_K = 8
LOAD_BALANCE_COEF = 0.001
BLOCK_T = 1024


def _router_block(x_ref, wt_ref, b_ref, rw_ref, se_ref, aux_ref,
                  accf_ref, accp_ref, *, n_tokens, n_blocks):
    i = pl.program_id(0)
    xb = x_ref[...].astype(jnp.bfloat16)
    wt = wt_ref[...]
    logits = jnp.dot(xb, wt, preferred_element_type=jnp.float32)
    logits = logits + b_ref[...]

    t = logits.shape[0]
    lt = logits.T
    eiota = jax.lax.broadcasted_iota(jnp.int32, (NUM_EXPERTS, t), 0)

    running = lt
    vals = []
    idxs = []
    for _ in range(TOP_K):
        m = jnp.max(running, axis=0, keepdims=True)
        cand = jnp.where(running == m, eiota, NUM_EXPERTS)
        sel = jnp.min(cand, axis=0, keepdims=True)
        vals.append(m)
        idxs.append(sel)
        running = jnp.where(eiota == sel, -jnp.inf, running)

    top_vals = jnp.concatenate(vals, axis=0)
    top_idx = jnp.concatenate(idxs, axis=0)
    m0 = vals[0]

    e_top = jnp.exp(top_vals - m0)
    rw = e_top / jnp.sum(e_top, axis=0, keepdims=True)
    rw_ref[...] = rw.T
    se_ref[...] = top_idx.T

    e_all = jnp.exp(lt - m0)
    probs = e_all / jnp.sum(e_all, axis=0, keepdims=True)
    p_part = jnp.sum(probs, axis=1, keepdims=True)
    f_part = jnp.sum(jnp.where(eiota == idxs[0], 1.0, 0.0),
                     axis=1, keepdims=True)

    @pl.when(i == 0)
    def _():
        accf_ref[...] = f_part
        accp_ref[...] = p_part

    @pl.when(i > 0)
    def _():
        accf_ref[...] += f_part
        accp_ref[...] += p_part

    @pl.when(i == n_blocks - 1)
    def _():
        scale = NUM_EXPERTS * LOAD_BALANCE_COEF / (n_tokens * n_tokens)
        aux_ref[...] = (scale * jnp.sum(accf_ref[...] * accp_ref[...])
                        ).reshape(1, 1)


def kernel(x, W, b):
    bsz, seq, hidden = x.shape
    n_tokens = bsz * seq
    x2 = x.reshape(n_tokens, hidden)
    wt = W.T.astype(jnp.bfloat16)
    b2 = b.reshape(1, NUM_EXPERTS)
    n_blocks = n_tokens // BLOCK_T

    body = functools.partial(_router_block, n_tokens=n_tokens,
                             n_blocks=n_blocks)
    rw, se, aux = pl.pallas_call(
        body,
        grid=(n_blocks,),
        in_specs=[
            pl.BlockSpec((BLOCK_T, hidden), lambda i: (i, 0)),
            pl.BlockSpec((hidden, NUM_EXPERTS), lambda i: (0, 0)),
            pl.BlockSpec((1, NUM_EXPERTS), lambda i: (0, 0)),
        ],
        out_specs=[
            pl.BlockSpec((BLOCK_T, TOP_K), lambda i: (i, 0)),
            pl.BlockSpec((BLOCK_T, TOP_K), lambda i: (i, 0)),
            pl.BlockSpec((1, 1), lambda i: (0, 0)),
        ],
        out_shape=[
            jax.ShapeDtypeStruct((n_tokens, TOP_K), jnp.float32),
            jax.ShapeDtypeStruct((n_tokens, TOP_K), jnp.int32),
            jax.ShapeDtypeStruct((1, 1), jnp.float32),
        ],
        scratch_shapes=[
            pltpu.VMEM((NUM_EXPERTS, 1), jnp.float32),
            pltpu.VMEM((NUM_EXPERTS, 1), jnp.float32),
        ],
    )(x2, wt, b2)

    return (rw.reshape(bsz, seq, TOP_K),
            se.reshape(bsz, seq, TOP_K),
            aux.reshape(()))
